# Initial kernel scaffold; baseline (speedup 1.0000x reference)
import jax
import jax.numpy as jnp
from jax.experimental import pallas as pl


def kernel(x, edge_index, eps, W1, b1, W2, b2, Wl, bl):
    raise NotImplementedError("write your pallas kernel here")


# speedup vs baseline: 11.9622x; 11.9622x over previous
import functools

import jax
import jax.numpy as jnp
import numpy as np
from jax.experimental import pallas as pl
from jax.experimental.pallas import tpu as pltpu

N = 10000
E = 20000


def _mm_kernel(a_ref, b_ref, o_ref):
    o_ref[...] = jnp.dot(a_ref[...], b_ref[...],
                         preferred_element_type=jnp.float32)


def _matmul(a, b, bm):
    m, k = a.shape
    _, n = b.shape
    return pl.pallas_call(
        _mm_kernel,
        grid=(m // bm,),
        in_specs=[
            pl.BlockSpec((bm, k), lambda i: (i, np.int32(0))),
            pl.BlockSpec((k, n), lambda i: (np.int32(0), np.int32(0))),
        ],
        out_specs=pl.BlockSpec((bm, n), lambda i: (i, np.int32(0))),
        out_shape=jax.ShapeDtypeStruct((m, n), jnp.float32),
    )(a, b)


def _i32loop(lo, hi, body):
    def wrapped(i, carry):
        body(i)
        return carry
    jax.lax.fori_loop(np.int32(lo), np.int32(hi), wrapped, jnp.int32(0))


def _edge1_kernel(idx_ref, xt_ref, xb_ref, b1_ref, out_ref, dinv_out_ref,
                  acc_ref, dinv_ref):
    j = pl.program_id(0)

    @pl.when(j == 0)
    def _():
        def init_i(i):
            dinv_ref[i] = 1.0
        _i32loop(0, N, init_i)

        def deg_e(e):
            c = idx_ref[1, e]
            dinv_ref[c] += 1.0
        _i32loop(0, E, deg_e)

        def rsq_i(i):
            d = 1.0 / jnp.sqrt(dinv_ref[i])
            dinv_ref[i] = d
            dinv_out_ref[i] = d
        _i32loop(0, N, rsq_i)

    def loop_a_lo(e):
        r = idx_ref[0, e]
        c = idx_ref[1, e]
        v = xt_ref[pl.ds(r, 1), :] + xb_ref[pl.ds(c, 1), :]
        out_ref[pl.ds(e, 1), :] = v
        de = dinv_ref[e]
        acc_ref[pl.ds(e, 1), :] = v * (de * de)
    _i32loop(0, N, loop_a_lo)

    def loop_a_hi(e):
        r = idx_ref[0, e]
        c = idx_ref[1, e]
        out_ref[pl.ds(e, 1), :] = (xt_ref[pl.ds(r, 1), :]
                                   + xb_ref[pl.ds(c, 1), :])
    _i32loop(N, E, loop_a_hi)

    def loop_b(e):
        r = idx_ref[0, e]
        c = idx_ref[1, e]
        nrm = dinv_ref[r] * dinv_ref[c]
        acc_ref[pl.ds(c, 1), :] += out_ref[pl.ds(r, 1), :] * nrm
    _i32loop(0, E, loop_b)

    out_ref[pl.ds(0, N), :] = jnp.maximum(acc_ref[...] + b1_ref[...], 0.0)
    out_ref[pl.ds(N, N), :] = jnp.maximum(
        out_ref[pl.ds(N, N), :] + b1_ref[...], 0.0)


def _edge2_kernel(idx_ref, h2_ref, dinv_smem_ref, b2_ref, wl_ref, bl_ref,
                  eps_ref, mask_ref, kld_ref, acc_ref):
    def loop_a_lo(e):
        de = dinv_smem_ref[e]
        acc_ref[pl.ds(e, 1), :] = h2_ref[pl.ds(e, 1), :] * (de * de)
    _i32loop(0, N, loop_a_lo)

    def loop_b(e):
        r = idx_ref[0, e]
        c = idx_ref[1, e]
        nrm = dinv_smem_ref[r] * dinv_smem_ref[c]
        acc_ref[pl.ds(c, 1), :] += h2_ref[pl.ds(r, 1), :] * nrm
    _i32loop(0, E, loop_b)

    b2 = b2_ref[...]
    wl = wl_ref[...]
    bl = bl_ref[0, 0]

    lo = jnp.maximum(acc_ref[...] + b2, 0.0)
    z_lo = jnp.sum(lo * wl, axis=1, keepdims=True) + bl
    hi = jnp.maximum(h2_ref[pl.ds(N, N), :] + b2, 0.0)
    z_hi = jnp.sum(hi * wl, axis=1, keepdims=True) + bl

    eps_lo = eps_ref[pl.ds(0, N), :]
    eps_hi = eps_ref[pl.ds(N, N), :]
    g_lo = jnp.log(eps_lo) - jnp.log(1.0 - eps_lo) + z_lo
    g_hi = jnp.log(eps_hi) - jnp.log(1.0 - eps_hi) + z_hi
    m_lo = 1.0 / (1.0 + jnp.exp(-g_lo))
    m_hi = 1.0 / (1.0 + jnp.exp(-g_hi))
    mask_ref[pl.ds(0, N), :] = m_lo
    mask_ref[pl.ds(N, N), :] = m_hi

    def kterm(m):
        return m * jnp.log(2.0 * m + 1e-08) \
            + (1.0 - m) * jnp.log(2.0 * (1.0 - m) + 1e-09)

    kld = (jnp.sum(kterm(m_lo)) + jnp.sum(kterm(m_hi))) / jnp.float32(E)
    kld_ref[...] = kld.reshape(1, 1)


@jax.jit
def kernel(x, edge_index, eps, W1, b1, W2, b2, Wl, bl):
    with jax.enable_x64(False):
        kld, mask = _pipeline(x, edge_index, eps, W1, b1, W2, b2, Wl, bl)
    odt = W1.dtype if jnp.issubdtype(W1.dtype, jnp.floating) else jnp.float32
    return (kld[0, 0].astype(odt), mask.astype(odt))


def _pipeline(x, edge_index, eps, W1, b1, W2, b2, Wl, bl):
    x = x.astype(jnp.float32)
    idx = edge_index.astype(jnp.int32)
    eps = eps.astype(jnp.float32)
    d = x.shape[1]
    f1 = W1.shape[1]
    f2 = W2.shape[1]

    w1cat = jnp.concatenate(
        [W1[:d].astype(jnp.float32), W1[d:].astype(jnp.float32)], axis=1)
    xtb = _matmul(x, w1cat, bm=400)

    nblk = 4
    fh = f1 // nblk
    out1, dinv = pl.pallas_call(
        _edge1_kernel,
        grid=(nblk,),
        in_specs=[
            pl.BlockSpec(memory_space=pltpu.SMEM),
            pl.BlockSpec((N, fh), lambda j: (np.int32(0), j)),
            pl.BlockSpec((N, fh), lambda j: (np.int32(0), j + np.int32(nblk))),
            pl.BlockSpec((1, fh), lambda j: (np.int32(0), j)),
        ],
        out_specs=[
            pl.BlockSpec((E, fh), lambda j: (np.int32(0), j)),
            pl.BlockSpec(memory_space=pltpu.SMEM),
        ],
        out_shape=[
            jax.ShapeDtypeStruct((E, f1), jnp.float32),
            jax.ShapeDtypeStruct((N,), jnp.float32),
        ],
        scratch_shapes=[
            pltpu.VMEM((N, fh), jnp.float32),
            pltpu.SMEM((N,), jnp.float32),
        ],
    )(idx, xtb, xtb, b1.astype(jnp.float32).reshape(1, f1))

    h2 = _matmul(out1, W2.astype(jnp.float32), bm=400)

    mask, kld = pl.pallas_call(
        _edge2_kernel,
        grid=(1,),
        in_specs=[
            pl.BlockSpec(memory_space=pltpu.SMEM),
            pl.BlockSpec((E, f2), lambda j: (np.int32(0), np.int32(0))),
            pl.BlockSpec(memory_space=pltpu.SMEM),
            pl.BlockSpec((1, f2), lambda j: (np.int32(0), np.int32(0))),
            pl.BlockSpec((1, f2), lambda j: (np.int32(0), np.int32(0))),
            pl.BlockSpec((1, 1), lambda j: (np.int32(0), np.int32(0))),
            pl.BlockSpec((E, 1), lambda j: (np.int32(0), np.int32(0))),
        ],
        out_specs=[
            pl.BlockSpec((E, 1), lambda j: (np.int32(0), np.int32(0))),
            pl.BlockSpec((1, 1), lambda j: (np.int32(0), np.int32(0))),
        ],
        out_shape=[
            jax.ShapeDtypeStruct((E, 1), jnp.float32),
            jax.ShapeDtypeStruct((1, 1), jnp.float32),
        ],
        scratch_shapes=[
            pltpu.VMEM((N, f2), jnp.float32),
        ],
    )(idx, h2, dinv, b2.astype(jnp.float32).reshape(1, f2),
      Wl.astype(jnp.float32).reshape(1, f2),
      bl.astype(jnp.float32).reshape(1, 1), eps)

    return kld, mask


# speedup vs baseline: 17.7838x; 1.4867x over previous
import functools

import jax
import jax.numpy as jnp
import numpy as np
from jax.experimental import pallas as pl
from jax.experimental.pallas import tpu as pltpu

N = 10000
E = 20000


def _mm_kernel(a_ref, b_ref, o_ref):
    o_ref[...] = jnp.dot(a_ref[...], b_ref[...],
                         preferred_element_type=jnp.float32)


def _matmul(a, b, bm):
    m, k = a.shape
    _, n = b.shape
    return pl.pallas_call(
        _mm_kernel,
        grid=(m // bm,),
        in_specs=[
            pl.BlockSpec((bm, k), lambda i: (i, np.int32(0))),
            pl.BlockSpec((k, n), lambda i: (np.int32(0), np.int32(0))),
        ],
        out_specs=pl.BlockSpec((bm, n), lambda i: (i, np.int32(0))),
        out_shape=jax.ShapeDtypeStruct((m, n), jnp.float32),
    )(a, b)


def _i32loop(lo, hi, body, unroll=1):
    def wrapped(i, carry):
        body(i)
        return carry
    jax.lax.fori_loop(np.int32(lo), np.int32(hi), wrapped, jnp.int32(0),
                      unroll=unroll)


def _edge1_kernel(idx_ref, xt_ref, xb_ref, b1_ref, out_ref, dinv_out_ref,
                  acc_ref, dinv_ref):
    j = pl.program_id(0)

    @pl.when(j == 0)
    def _():
        def init_i(i):
            dinv_ref[i] = 1.0
        _i32loop(0, N, init_i)

        def deg_e(e):
            c = idx_ref[1, e]
            dinv_ref[c] += 1.0
        _i32loop(0, E, deg_e)

        def rsq_i(i):
            d = 1.0 / jnp.sqrt(dinv_ref[i])
            dinv_ref[i] = d
            dinv_out_ref[i] = d
        _i32loop(0, N, rsq_i)

    def loop_a_lo(e):
        r = idx_ref[0, e]
        c = idx_ref[1, e]
        v = xt_ref[pl.ds(r, 1), :] + xb_ref[pl.ds(c, 1), :]
        out_ref[pl.ds(e, 1), :] = v
        de = dinv_ref[e]
        acc_ref[pl.ds(e, 1), :] = v * (de * de)
    _i32loop(0, N, loop_a_lo, unroll=8)

    def loop_a_hi(e):
        r = idx_ref[0, e]
        c = idx_ref[1, e]
        out_ref[pl.ds(e, 1), :] = (xt_ref[pl.ds(r, 1), :]
                                   + xb_ref[pl.ds(c, 1), :])
    _i32loop(N, E, loop_a_hi, unroll=8)

    def loop_b(e):
        r = idx_ref[0, e]
        c = idx_ref[1, e]
        nrm = dinv_ref[r] * dinv_ref[c]
        acc_ref[pl.ds(c, 1), :] += out_ref[pl.ds(r, 1), :] * nrm
    _i32loop(0, E, loop_b, unroll=8)

    out_ref[pl.ds(0, N), :] = jnp.maximum(acc_ref[...] + b1_ref[...], 0.0)
    out_ref[pl.ds(N, N), :] = jnp.maximum(
        out_ref[pl.ds(N, N), :] + b1_ref[...], 0.0)


def _edge2_kernel(idx_ref, h2_ref, dinv_smem_ref, b2_ref, wl_ref, bl_ref,
                  eps_ref, mask_ref, kld_ref, acc_ref):
    def loop_a_lo(e):
        de = dinv_smem_ref[e]
        acc_ref[pl.ds(e, 1), :] = h2_ref[pl.ds(e, 1), :] * (de * de)
    _i32loop(0, N, loop_a_lo, unroll=8)

    def loop_b(e):
        r = idx_ref[0, e]
        c = idx_ref[1, e]
        nrm = dinv_smem_ref[r] * dinv_smem_ref[c]
        acc_ref[pl.ds(c, 1), :] += h2_ref[pl.ds(r, 1), :] * nrm
    _i32loop(0, E, loop_b, unroll=8)

    b2 = b2_ref[...]
    wl = wl_ref[...]
    bl = bl_ref[0, 0]

    lo = jnp.maximum(acc_ref[...] + b2, 0.0)
    z_lo = jnp.sum(lo * wl, axis=1, keepdims=True) + bl
    hi = jnp.maximum(h2_ref[pl.ds(N, N), :] + b2, 0.0)
    z_hi = jnp.sum(hi * wl, axis=1, keepdims=True) + bl

    eps_lo = eps_ref[pl.ds(0, N), :]
    eps_hi = eps_ref[pl.ds(N, N), :]
    g_lo = jnp.log(eps_lo) - jnp.log(1.0 - eps_lo) + z_lo
    g_hi = jnp.log(eps_hi) - jnp.log(1.0 - eps_hi) + z_hi
    m_lo = 1.0 / (1.0 + jnp.exp(-g_lo))
    m_hi = 1.0 / (1.0 + jnp.exp(-g_hi))
    mask_ref[pl.ds(0, N), :] = m_lo
    mask_ref[pl.ds(N, N), :] = m_hi

    def kterm(m):
        return m * jnp.log(2.0 * m + 1e-08) \
            + (1.0 - m) * jnp.log(2.0 * (1.0 - m) + 1e-09)

    kld = (jnp.sum(kterm(m_lo)) + jnp.sum(kterm(m_hi))) / jnp.float32(E)
    kld_ref[...] = kld.reshape(1, 1)


@jax.jit
def kernel(x, edge_index, eps, W1, b1, W2, b2, Wl, bl):
    with jax.enable_x64(False):
        kld, mask = _pipeline(x, edge_index, eps, W1, b1, W2, b2, Wl, bl)
    odt = W1.dtype if jnp.issubdtype(W1.dtype, jnp.floating) else jnp.float32
    return (kld[0, 0].astype(odt), mask.astype(odt))


def _pipeline(x, edge_index, eps, W1, b1, W2, b2, Wl, bl):
    x = x.astype(jnp.float32)
    idx = edge_index.astype(jnp.int32)
    eps = eps.astype(jnp.float32)
    d = x.shape[1]
    f1 = W1.shape[1]
    f2 = W2.shape[1]

    w1cat = jnp.concatenate(
        [W1[:d].astype(jnp.float32), W1[d:].astype(jnp.float32)], axis=1)
    xtb = _matmul(x, w1cat, bm=400)

    nblk = 4
    fh = f1 // nblk
    out1, dinv = pl.pallas_call(
        _edge1_kernel,
        grid=(nblk,),
        in_specs=[
            pl.BlockSpec(memory_space=pltpu.SMEM),
            pl.BlockSpec((N, fh), lambda j: (np.int32(0), j)),
            pl.BlockSpec((N, fh), lambda j: (np.int32(0), j + np.int32(nblk))),
            pl.BlockSpec((1, fh), lambda j: (np.int32(0), j)),
        ],
        out_specs=[
            pl.BlockSpec((E, fh), lambda j: (np.int32(0), j)),
            pl.BlockSpec(memory_space=pltpu.SMEM),
        ],
        out_shape=[
            jax.ShapeDtypeStruct((E, f1), jnp.float32),
            jax.ShapeDtypeStruct((N,), jnp.float32),
        ],
        scratch_shapes=[
            pltpu.VMEM((N, fh), jnp.float32),
            pltpu.SMEM((N,), jnp.float32),
        ],
    )(idx, xtb, xtb, b1.astype(jnp.float32).reshape(1, f1))

    h2 = _matmul(out1, W2.astype(jnp.float32), bm=400)

    mask, kld = pl.pallas_call(
        _edge2_kernel,
        grid=(1,),
        in_specs=[
            pl.BlockSpec(memory_space=pltpu.SMEM),
            pl.BlockSpec((E, f2), lambda j: (np.int32(0), np.int32(0))),
            pl.BlockSpec(memory_space=pltpu.SMEM),
            pl.BlockSpec((1, f2), lambda j: (np.int32(0), np.int32(0))),
            pl.BlockSpec((1, f2), lambda j: (np.int32(0), np.int32(0))),
            pl.BlockSpec((1, 1), lambda j: (np.int32(0), np.int32(0))),
            pl.BlockSpec((E, 1), lambda j: (np.int32(0), np.int32(0))),
        ],
        out_specs=[
            pl.BlockSpec((E, 1), lambda j: (np.int32(0), np.int32(0))),
            pl.BlockSpec((1, 1), lambda j: (np.int32(0), np.int32(0))),
        ],
        out_shape=[
            jax.ShapeDtypeStruct((E, 1), jnp.float32),
            jax.ShapeDtypeStruct((1, 1), jnp.float32),
        ],
        scratch_shapes=[
            pltpu.VMEM((N, f2), jnp.float32),
        ],
    )(idx, h2, dinv, b2.astype(jnp.float32).reshape(1, f2),
      Wl.astype(jnp.float32).reshape(1, f2),
      bl.astype(jnp.float32).reshape(1, 1), eps)

    return kld, mask


# speedup vs baseline: 19.3031x; 1.0854x over previous
import functools

import jax
import jax.numpy as jnp
import numpy as np
from jax.experimental import pallas as pl
from jax.experimental.pallas import tpu as pltpu

N = 10000
E = 20000


def _mm_kernel(a_ref, b_ref, o_ref):
    o_ref[...] = jnp.dot(a_ref[...], b_ref[...],
                         preferred_element_type=jnp.float32)


def _matmul(a, b, bm):
    m, k = a.shape
    _, n = b.shape
    return pl.pallas_call(
        _mm_kernel,
        grid=(m // bm,),
        in_specs=[
            pl.BlockSpec((bm, k), lambda i: (i, np.int32(0))),
            pl.BlockSpec((k, n), lambda i: (np.int32(0), np.int32(0))),
        ],
        out_specs=pl.BlockSpec((bm, n), lambda i: (i, np.int32(0))),
        out_shape=jax.ShapeDtypeStruct((m, n), jnp.float32),
    )(a, b)


def _i32loop(lo, hi, body, unroll=1):
    def wrapped(i, carry):
        body(i)
        return carry
    jax.lax.fori_loop(np.int32(lo), np.int32(hi), wrapped, jnp.int32(0),
                      unroll=unroll)


def _edge1_kernel(idx_ref, xt_ref, xb_ref, b1_ref, out_ref, dinv_out_ref,
                  acc_ref, dinv_ref):
    j = pl.program_id(0)

    @pl.when(j == 0)
    def _():
        def init_i(i):
            dinv_ref[i] = 1.0
        _i32loop(0, N, init_i, unroll=8)

        def deg_e(e):
            c = idx_ref[1, e]
            dinv_ref[c] += 1.0
        _i32loop(0, E, deg_e, unroll=8)

        def rsq_i(i):
            d = 1.0 / jnp.sqrt(dinv_ref[i])
            dinv_ref[i] = d
            dinv_out_ref[i] = d
        _i32loop(0, N, rsq_i, unroll=8)

    def loop_a_lo(e):
        r = idx_ref[0, e]
        c = idx_ref[1, e]
        v = xt_ref[pl.ds(r, 1), :] + xb_ref[pl.ds(c, 1), :]
        out_ref[pl.ds(e, 1), :] = v
        de = dinv_ref[e]
        acc_ref[pl.ds(e, 1), :] = v * (de * de)
    _i32loop(0, N, loop_a_lo, unroll=16)

    def loop_a_hi(e):
        r = idx_ref[0, e]
        c = idx_ref[1, e]
        out_ref[pl.ds(e, 1), :] = (xt_ref[pl.ds(r, 1), :]
                                   + xb_ref[pl.ds(c, 1), :])
    _i32loop(N, E, loop_a_hi, unroll=16)

    def loop_b(e):
        r = idx_ref[0, e]
        c = idx_ref[1, e]
        nrm = dinv_ref[r] * dinv_ref[c]
        acc_ref[pl.ds(c, 1), :] += out_ref[pl.ds(r, 1), :] * nrm
    _i32loop(0, E, loop_b, unroll=16)

    out_ref[pl.ds(0, N), :] = jnp.maximum(acc_ref[...] + b1_ref[...], 0.0)
    out_ref[pl.ds(N, N), :] = jnp.maximum(
        out_ref[pl.ds(N, N), :] + b1_ref[...], 0.0)


def _edge2_kernel(idx_ref, h2_ref, dinv_smem_ref, b2_ref, wl_ref, bl_ref,
                  eps_ref, mask_ref, kld_ref, acc_ref):
    def loop_a_lo(e):
        de = dinv_smem_ref[e]
        acc_ref[pl.ds(e, 1), :] = h2_ref[pl.ds(e, 1), :] * (de * de)
    _i32loop(0, N, loop_a_lo, unroll=16)

    def loop_b(e):
        r = idx_ref[0, e]
        c = idx_ref[1, e]
        nrm = dinv_smem_ref[r] * dinv_smem_ref[c]
        acc_ref[pl.ds(c, 1), :] += h2_ref[pl.ds(r, 1), :] * nrm
    _i32loop(0, E, loop_b, unroll=16)

    b2 = b2_ref[...]
    wl = wl_ref[...]
    bl = bl_ref[0, 0]

    lo = jnp.maximum(acc_ref[...] + b2, 0.0)
    z_lo = jnp.sum(lo * wl, axis=1, keepdims=True) + bl
    hi = jnp.maximum(h2_ref[pl.ds(N, N), :] + b2, 0.0)
    z_hi = jnp.sum(hi * wl, axis=1, keepdims=True) + bl

    eps_lo = eps_ref[pl.ds(0, N), :]
    eps_hi = eps_ref[pl.ds(N, N), :]
    g_lo = jnp.log(eps_lo) - jnp.log(1.0 - eps_lo) + z_lo
    g_hi = jnp.log(eps_hi) - jnp.log(1.0 - eps_hi) + z_hi
    m_lo = 1.0 / (1.0 + jnp.exp(-g_lo))
    m_hi = 1.0 / (1.0 + jnp.exp(-g_hi))
    mask_ref[pl.ds(0, N), :] = m_lo
    mask_ref[pl.ds(N, N), :] = m_hi

    def kterm(m):
        return m * jnp.log(2.0 * m + 1e-08) \
            + (1.0 - m) * jnp.log(2.0 * (1.0 - m) + 1e-09)

    kld = (jnp.sum(kterm(m_lo)) + jnp.sum(kterm(m_hi))) / jnp.float32(E)
    kld_ref[...] = kld.reshape(1, 1)


@jax.jit
def kernel(x, edge_index, eps, W1, b1, W2, b2, Wl, bl):
    with jax.enable_x64(False):
        kld, mask = _pipeline(x, edge_index, eps, W1, b1, W2, b2, Wl, bl)
    odt = W1.dtype if jnp.issubdtype(W1.dtype, jnp.floating) else jnp.float32
    return (kld[0, 0].astype(odt), mask.astype(odt))


def _pipeline(x, edge_index, eps, W1, b1, W2, b2, Wl, bl):
    x = x.astype(jnp.float32)
    idx = edge_index.astype(jnp.int32)
    eps = eps.astype(jnp.float32)
    d = x.shape[1]
    f1 = W1.shape[1]
    f2 = W2.shape[1]

    w1cat = jnp.concatenate(
        [W1[:d].astype(jnp.float32), W1[d:].astype(jnp.float32)], axis=1)
    xtb = _matmul(x, w1cat, bm=400)

    nblk = 4
    fh = f1 // nblk
    out1, dinv = pl.pallas_call(
        _edge1_kernel,
        grid=(nblk,),
        in_specs=[
            pl.BlockSpec(memory_space=pltpu.SMEM),
            pl.BlockSpec((N, fh), lambda j: (np.int32(0), j)),
            pl.BlockSpec((N, fh), lambda j: (np.int32(0), j + np.int32(nblk))),
            pl.BlockSpec((1, fh), lambda j: (np.int32(0), j)),
        ],
        out_specs=[
            pl.BlockSpec((E, fh), lambda j: (np.int32(0), j)),
            pl.BlockSpec(memory_space=pltpu.SMEM),
        ],
        out_shape=[
            jax.ShapeDtypeStruct((E, f1), jnp.float32),
            jax.ShapeDtypeStruct((N,), jnp.float32),
        ],
        scratch_shapes=[
            pltpu.VMEM((N, fh), jnp.float32),
            pltpu.SMEM((N,), jnp.float32),
        ],
    )(idx, xtb, xtb, b1.astype(jnp.float32).reshape(1, f1))

    h2 = _matmul(out1, W2.astype(jnp.float32), bm=400)

    mask, kld = pl.pallas_call(
        _edge2_kernel,
        grid=(1,),
        in_specs=[
            pl.BlockSpec(memory_space=pltpu.SMEM),
            pl.BlockSpec((E, f2), lambda j: (np.int32(0), np.int32(0))),
            pl.BlockSpec(memory_space=pltpu.SMEM),
            pl.BlockSpec((1, f2), lambda j: (np.int32(0), np.int32(0))),
            pl.BlockSpec((1, f2), lambda j: (np.int32(0), np.int32(0))),
            pl.BlockSpec((1, 1), lambda j: (np.int32(0), np.int32(0))),
            pl.BlockSpec((E, 1), lambda j: (np.int32(0), np.int32(0))),
        ],
        out_specs=[
            pl.BlockSpec((E, 1), lambda j: (np.int32(0), np.int32(0))),
            pl.BlockSpec((1, 1), lambda j: (np.int32(0), np.int32(0))),
        ],
        out_shape=[
            jax.ShapeDtypeStruct((E, 1), jnp.float32),
            jax.ShapeDtypeStruct((1, 1), jnp.float32),
        ],
        scratch_shapes=[
            pltpu.VMEM((N, f2), jnp.float32),
        ],
    )(idx, h2, dinv, b2.astype(jnp.float32).reshape(1, f2),
      Wl.astype(jnp.float32).reshape(1, f2),
      bl.astype(jnp.float32).reshape(1, 1), eps)

    return kld, mask


# speedup vs baseline: 19.4133x; 1.0057x over previous
import functools

import jax
import jax.numpy as jnp
import numpy as np
from jax.experimental import pallas as pl
from jax.experimental.pallas import tpu as pltpu

N = 10000
E = 20000


def _mm_kernel(a_ref, b_ref, o_ref):
    o_ref[...] = jnp.dot(a_ref[...], b_ref[...],
                         preferred_element_type=jnp.float32)


def _matmul(a, b, bm):
    m, k = a.shape
    _, n = b.shape
    return pl.pallas_call(
        _mm_kernel,
        grid=(m // bm,),
        in_specs=[
            pl.BlockSpec((bm, k), lambda i: (i, np.int32(0))),
            pl.BlockSpec((k, n), lambda i: (np.int32(0), np.int32(0))),
        ],
        out_specs=pl.BlockSpec((bm, n), lambda i: (i, np.int32(0))),
        out_shape=jax.ShapeDtypeStruct((m, n), jnp.float32),
    )(a, b)


def _i32loop(lo, hi, body, unroll=1):
    def wrapped(i, carry):
        body(i)
        return carry
    jax.lax.fori_loop(np.int32(lo), np.int32(hi), wrapped, jnp.int32(0),
                      unroll=unroll)


def _edge1_kernel(idx_ref, xt_ref, xb_ref, b1_ref, out_ref, dinv_out_ref,
                  acc_ref, dinv_ref):
    j = pl.program_id(0)

    @pl.when(j == 0)
    def _():
        def init_i(i):
            dinv_ref[i] = 1.0
        _i32loop(0, N, init_i, unroll=8)

        def deg_e(e):
            c = idx_ref[1, e]
            dinv_ref[c] += 1.0
        _i32loop(0, E, deg_e, unroll=8)

        def rsq_i(i):
            d = 1.0 / jnp.sqrt(dinv_ref[i])
            dinv_ref[i] = d
            dinv_out_ref[i] = d
        _i32loop(0, N, rsq_i, unroll=8)

    def loop_a_lo(e):
        r = idx_ref[0, e]
        c = idx_ref[1, e]
        v = xt_ref[pl.ds(r, 1), :] + xb_ref[pl.ds(c, 1), :]
        out_ref[pl.ds(e, 1), :] = v
        de = dinv_ref[e]
        acc_ref[pl.ds(e, 1), :] = v * (de * de)
    _i32loop(0, N, loop_a_lo, unroll=32)

    def loop_a_hi(e):
        r = idx_ref[0, e]
        c = idx_ref[1, e]
        out_ref[pl.ds(e, 1), :] = (xt_ref[pl.ds(r, 1), :]
                                   + xb_ref[pl.ds(c, 1), :])
    _i32loop(N, E, loop_a_hi, unroll=32)

    def loop_b(e):
        r = idx_ref[0, e]
        c = idx_ref[1, e]
        nrm = dinv_ref[r] * dinv_ref[c]
        acc_ref[pl.ds(c, 1), :] += out_ref[pl.ds(r, 1), :] * nrm
    _i32loop(0, E, loop_b, unroll=32)

    out_ref[pl.ds(0, N), :] = jnp.maximum(acc_ref[...] + b1_ref[...], 0.0)
    out_ref[pl.ds(N, N), :] = jnp.maximum(
        out_ref[pl.ds(N, N), :] + b1_ref[...], 0.0)


def _edge2_kernel(idx_ref, h2_ref, dinv_smem_ref, b2_ref, wl_ref, bl_ref,
                  eps_ref, mask_ref, kld_ref, acc_ref):
    def loop_a_lo(e):
        de = dinv_smem_ref[e]
        acc_ref[pl.ds(e, 1), :] = h2_ref[pl.ds(e, 1), :] * (de * de)
    _i32loop(0, N, loop_a_lo, unroll=32)

    def loop_b(e):
        r = idx_ref[0, e]
        c = idx_ref[1, e]
        nrm = dinv_smem_ref[r] * dinv_smem_ref[c]
        acc_ref[pl.ds(c, 1), :] += h2_ref[pl.ds(r, 1), :] * nrm
    _i32loop(0, E, loop_b, unroll=32)

    b2 = b2_ref[...]
    wl = wl_ref[...]
    bl = bl_ref[0, 0]

    lo = jnp.maximum(acc_ref[...] + b2, 0.0)
    z_lo = jnp.sum(lo * wl, axis=1, keepdims=True) + bl
    hi = jnp.maximum(h2_ref[pl.ds(N, N), :] + b2, 0.0)
    z_hi = jnp.sum(hi * wl, axis=1, keepdims=True) + bl

    eps_lo = eps_ref[pl.ds(0, N), :]
    eps_hi = eps_ref[pl.ds(N, N), :]
    g_lo = jnp.log(eps_lo) - jnp.log(1.0 - eps_lo) + z_lo
    g_hi = jnp.log(eps_hi) - jnp.log(1.0 - eps_hi) + z_hi
    m_lo = 1.0 / (1.0 + jnp.exp(-g_lo))
    m_hi = 1.0 / (1.0 + jnp.exp(-g_hi))
    mask_ref[pl.ds(0, N), :] = m_lo
    mask_ref[pl.ds(N, N), :] = m_hi

    def kterm(m):
        return m * jnp.log(2.0 * m + 1e-08) \
            + (1.0 - m) * jnp.log(2.0 * (1.0 - m) + 1e-09)

    kld = (jnp.sum(kterm(m_lo)) + jnp.sum(kterm(m_hi))) / jnp.float32(E)
    kld_ref[...] = kld.reshape(1, 1)


@jax.jit
def kernel(x, edge_index, eps, W1, b1, W2, b2, Wl, bl):
    with jax.enable_x64(False):
        kld, mask = _pipeline(x, edge_index, eps, W1, b1, W2, b2, Wl, bl)
    odt = W1.dtype if jnp.issubdtype(W1.dtype, jnp.floating) else jnp.float32
    return (kld[0, 0].astype(odt), mask.astype(odt))


def _pipeline(x, edge_index, eps, W1, b1, W2, b2, Wl, bl):
    x = x.astype(jnp.float32)
    idx = edge_index.astype(jnp.int32)
    eps = eps.astype(jnp.float32)
    d = x.shape[1]
    f1 = W1.shape[1]
    f2 = W2.shape[1]

    w1cat = jnp.concatenate(
        [W1[:d].astype(jnp.float32), W1[d:].astype(jnp.float32)], axis=1)
    xtb = _matmul(x, w1cat, bm=400)

    nblk = 4
    fh = f1 // nblk
    out1, dinv = pl.pallas_call(
        _edge1_kernel,
        grid=(nblk,),
        in_specs=[
            pl.BlockSpec(memory_space=pltpu.SMEM),
            pl.BlockSpec((N, fh), lambda j: (np.int32(0), j)),
            pl.BlockSpec((N, fh), lambda j: (np.int32(0), j + np.int32(nblk))),
            pl.BlockSpec((1, fh), lambda j: (np.int32(0), j)),
        ],
        out_specs=[
            pl.BlockSpec((E, fh), lambda j: (np.int32(0), j)),
            pl.BlockSpec(memory_space=pltpu.SMEM),
        ],
        out_shape=[
            jax.ShapeDtypeStruct((E, f1), jnp.float32),
            jax.ShapeDtypeStruct((N,), jnp.float32),
        ],
        scratch_shapes=[
            pltpu.VMEM((N, fh), jnp.float32),
            pltpu.SMEM((N,), jnp.float32),
        ],
    )(idx, xtb, xtb, b1.astype(jnp.float32).reshape(1, f1))

    h2 = _matmul(out1, W2.astype(jnp.float32), bm=400)

    mask, kld = pl.pallas_call(
        _edge2_kernel,
        grid=(1,),
        in_specs=[
            pl.BlockSpec(memory_space=pltpu.SMEM),
            pl.BlockSpec((E, f2), lambda j: (np.int32(0), np.int32(0))),
            pl.BlockSpec(memory_space=pltpu.SMEM),
            pl.BlockSpec((1, f2), lambda j: (np.int32(0), np.int32(0))),
            pl.BlockSpec((1, f2), lambda j: (np.int32(0), np.int32(0))),
            pl.BlockSpec((1, 1), lambda j: (np.int32(0), np.int32(0))),
            pl.BlockSpec((E, 1), lambda j: (np.int32(0), np.int32(0))),
        ],
        out_specs=[
            pl.BlockSpec((E, 1), lambda j: (np.int32(0), np.int32(0))),
            pl.BlockSpec((1, 1), lambda j: (np.int32(0), np.int32(0))),
        ],
        out_shape=[
            jax.ShapeDtypeStruct((E, 1), jnp.float32),
            jax.ShapeDtypeStruct((1, 1), jnp.float32),
        ],
        scratch_shapes=[
            pltpu.VMEM((N, f2), jnp.float32),
        ],
    )(idx, h2, dinv, b2.astype(jnp.float32).reshape(1, f2),
      Wl.astype(jnp.float32).reshape(1, f2),
      bl.astype(jnp.float32).reshape(1, 1), eps)

    return kld, mask
